# baseline (speedup 1.0000x reference)
import functools

import jax
import jax.numpy as jnp
from jax import lax
from jax.experimental import pallas as pl
from jax.experimental.pallas import tpu as pltpu
from jax.experimental.pallas import tpu_sc as plsc

NC = 2
NS = 16
NW = NC * NS

IDX_MINOR = 128
SUB = 4
CHUNK = IDX_MINOR * SUB


def _make_gather(n_rows: int, dim: int):
  assert n_rows % (NW * CHUNK) == 0
  rows_per_w = n_rows // NW
  chunks_per_w = rows_per_w // CHUNK
  idx_rows_per_w = rows_per_w // IDX_MINOR

  mesh = plsc.VectorSubcoreMesh(
      core_axis_name="c", subcore_axis_name="s",
      num_cores=NC, num_subcores=NS)

  @functools.partial(
      pl.kernel,
      mesh=mesh,
      out_type=jax.ShapeDtypeStruct((n_rows, dim), jnp.float32),
      scratch_types=[
          pltpu.VMEM((SUB, IDX_MINOR), jnp.int32),
          pltpu.VMEM((CHUNK, dim), jnp.float32),
          pltpu.SemaphoreType.DMA,
      ],
      compiler_params=pltpu.CompilerParams(use_tc_tiling_on_sc=False),
  )
  def gather(idx_hbm, table_hbm, out_hbm, idx_v, rows_v, sem):
    wid = lax.axis_index("s") * NC + lax.axis_index("c")
    row_base = wid * rows_per_w
    idx_row_base = wid * idx_rows_per_w

    def body(ci, carry):
      pltpu.sync_copy(idx_hbm.at[pl.ds(idx_row_base + ci * SUB, SUB)], idx_v)
      copies = [
          pltpu.async_copy(
              table_hbm.at[idx_v.at[j]],
              rows_v.at[pl.ds(j * IDX_MINOR, IDX_MINOR)],
              sem,
          )
          for j in range(SUB)
      ]
      for c in copies:
        c.wait()
      pltpu.sync_copy(
          rows_v, out_hbm.at[pl.ds(row_base + ci * CHUNK, CHUNK)])
      return carry

    lax.fori_loop(0, chunks_per_w, body, 0)

  return gather


def kernel(input, table):
  Bn, Ln = input.shape
  _, dim = table.shape
  n_rows = Bn * Ln
  idx = jnp.transpose(input, (1, 0)).reshape(n_rows // IDX_MINOR, IDX_MINOR)
  out = _make_gather(n_rows, dim)(idx, table)
  return out.reshape(Ln, Bn, dim)


# speedup vs baseline: 1.0391x; 1.0391x over previous
import functools

import jax
import jax.numpy as jnp
from jax import lax
from jax.experimental import pallas as pl
from jax.experimental.pallas import tpu as pltpu
from jax.experimental.pallas import tpu_sc as plsc

NC = 2
NS = 16
NW = NC * NS

IDX_MINOR = 128
SUB = 4
CHUNK = IDX_MINOR * SUB


def _make_gather(n_rows: int, dim: int):
  assert n_rows % (NW * 2 * CHUNK) == 0
  rows_per_w = n_rows // NW
  chunks_per_w = rows_per_w // CHUNK
  idx_rows_per_w = rows_per_w // IDX_MINOR
  n2 = chunks_per_w // 2 - 1

  mesh = plsc.VectorSubcoreMesh(
      core_axis_name="c", subcore_axis_name="s",
      num_cores=NC, num_subcores=NS)

  @functools.partial(
      pl.kernel,
      mesh=mesh,
      out_type=jax.ShapeDtypeStruct((n_rows, dim), jnp.float32),
      scratch_types=[
          pltpu.VMEM((idx_rows_per_w, IDX_MINOR), jnp.int32),
          pltpu.VMEM((CHUNK, dim), jnp.float32),
          pltpu.VMEM((CHUNK, dim), jnp.float32),
          pltpu.SemaphoreType.DMA,
          pltpu.SemaphoreType.DMA,
          pltpu.SemaphoreType.DMA,
          pltpu.SemaphoreType.DMA,
      ],
      compiler_params=pltpu.CompilerParams(use_tc_tiling_on_sc=False),
  )
  def gather(idx_hbm, table_hbm, out_hbm, idx_v, rows_a, rows_b,
             gsem_a, gsem_b, ssem_a, ssem_b):
    wid = lax.axis_index("s") * NC + lax.axis_index("c")
    row_base = wid * rows_per_w
    idx_row_base = wid * idx_rows_per_w

    pltpu.sync_copy(
        idx_hbm.at[pl.ds(idx_row_base, idx_rows_per_w)], idx_v)

    def fire_gathers(chunk, rows_v, sem):
      for j in range(SUB):
        pltpu.async_copy(
            table_hbm.at[idx_v.at[chunk * SUB + j]],
            rows_v.at[pl.ds(j * IDX_MINOR, IDX_MINOR)],
            sem,
        )

    def wait_gathers(rows_v, sem):
      pltpu.make_async_copy(table_hbm.at[pl.ds(0, CHUNK)], rows_v, sem).wait()

    def fire_store(chunk, rows_v, sem):
      pltpu.async_copy(
          rows_v, out_hbm.at[pl.ds(row_base + chunk * CHUNK, CHUNK)], sem)

    def wait_store(rows_v, sem):
      pltpu.make_async_copy(rows_v, out_hbm.at[pl.ds(0, CHUNK)], sem).wait()

    fire_gathers(0, rows_a, gsem_a)
    fire_gathers(1, rows_b, gsem_b)

    def body(i2, carry):
      c0 = 2 * i2
      wait_gathers(rows_a, gsem_a)
      fire_store(c0, rows_a, ssem_a)
      wait_gathers(rows_b, gsem_b)
      fire_store(c0 + 1, rows_b, ssem_b)
      wait_store(rows_a, ssem_a)
      fire_gathers(c0 + 2, rows_a, gsem_a)
      wait_store(rows_b, ssem_b)
      fire_gathers(c0 + 3, rows_b, gsem_b)
      return carry

    lax.fori_loop(0, n2, body, 0)

    c0 = 2 * n2
    wait_gathers(rows_a, gsem_a)
    fire_store(c0, rows_a, ssem_a)
    wait_gathers(rows_b, gsem_b)
    fire_store(c0 + 1, rows_b, ssem_b)
    wait_store(rows_a, ssem_a)
    wait_store(rows_b, ssem_b)

  return gather


def kernel(input, table):
  Bn, Ln = input.shape
  _, dim = table.shape
  n_rows = Bn * Ln
  idx = jnp.transpose(input, (1, 0)).reshape(n_rows // IDX_MINOR, IDX_MINOR)
  out = _make_gather(n_rows, dim)(idx, table)
  return out.reshape(Ln, Bn, dim)
